# baseline (speedup 1.0000x reference)
import functools

import jax
import jax.numpy as jnp
from jax.experimental import pallas as pl
from jax.experimental.pallas import tpu as pltpu

_KEYS = (
    (1832780943, 270669613),
    (64467757, 2916123636),
    (2465931498, 255383827),
    (3134548294, 894150801),
)
_THRESH = (2516583, 1677722, 3355444, 838861)

_ROTS = ((13, 15, 26, 6), (17, 29, 16, 24))


def _threefry_bits(cnt_u32, k0, k1):
    ks0 = jnp.uint32(k0)
    ks1 = jnp.uint32(k1)
    ks2 = jnp.uint32((k0 ^ k1 ^ 0x1BD11BDA) & 0xFFFFFFFF)
    ks = (ks0, ks1, ks2)
    x0 = jnp.full_like(cnt_u32, ks0)
    x1 = cnt_u32 + ks1
    for i in range(5):
        rots = _ROTS[i % 2]
        for r in rots:
            x0 = x0 + x1
            x1 = (x1 << jnp.uint32(r)) | (x1 >> jnp.uint32(32 - r))
            x1 = x1 ^ x0
        x0 = x0 + ks[(i + 1) % 3]
        x1 = x1 + ks[(i + 2) % 3] + jnp.uint32(i + 1)
    return x0 ^ x1


def _masks_for(cnt_u32):
    out = []
    for (k0, k1), t in zip(_KEYS, _THRESH):
        bits = _threefry_bits(cnt_u32, k0, k1)
        mant = jnp.right_shift(bits, jnp.uint32(9)).astype(jnp.int32)
        out.append(mant < t)
    return out


def _attrs_kernel(rows_per_blk, lanes, uid_ref, ts_ref, et_ref,
                  muid_ref, mts_ref, met_ref,
                  m0_ref, m1_ref, m2_ref, m3_ref):
    b = pl.program_id(0)
    shape = uid_ref.shape
    s_io = jax.lax.broadcasted_iota(jnp.int32, shape, 1)
    l_io = jax.lax.broadcasted_iota(jnp.int32, shape, 2)
    j = b * (rows_per_blk * lanes) + s_io * lanes + l_io
    m_uid, m_exe, m_ts, m_et = _masks_for(j.astype(jnp.uint32))
    muid_ref[...] = jnp.where(m_uid, 0, uid_ref[...])
    mts_ref[...] = jnp.where(m_ts, jnp.float32(0), ts_ref[...])
    met_ref[...] = jnp.where(m_et, 0, et_ref[...])
    m0_ref[...] = m_uid.astype(m0_ref.dtype)
    m1_ref[...] = m_exe.astype(m1_ref.dtype)
    m2_ref[...] = m_ts.astype(m2_ref.dtype)
    m3_ref[...] = m_et.astype(m3_ref.dtype)


def _exe_kernel(x_ref, m_ref, o_ref):
    m = m_ref[...]
    o_ref[...] = jnp.where(m != 0, jnp.float32(0), x_ref[...])


def kernel(uid, exe_path, timestamp, edge_type):
    n, d = exe_path.shape
    S, L = 8, 625
    G = n // (S * L)
    assert G * S * L == n

    uid3 = uid.reshape(G, S, L)
    ts3 = timestamp.reshape(G, S, L)
    et3 = edge_type.reshape(G, S, L)

    blk = pl.BlockSpec((1, S, L), lambda b: (b, 0, 0))
    stage_a = pl.pallas_call(
        functools.partial(_attrs_kernel, S, L),
        grid=(G,),
        in_specs=[blk, blk, blk],
        out_specs=[blk] * 7,
        out_shape=[
            jax.ShapeDtypeStruct((G, S, L), uid.dtype),
            jax.ShapeDtypeStruct((G, S, L), timestamp.dtype),
            jax.ShapeDtypeStruct((G, S, L), edge_type.dtype),
            jax.ShapeDtypeStruct((G, S, L), jnp.int8),
            jax.ShapeDtypeStruct((G, S, L), jnp.int32),
            jax.ShapeDtypeStruct((G, S, L), jnp.int8),
            jax.ShapeDtypeStruct((G, S, L), jnp.int8),
        ],
        compiler_params=pltpu.CompilerParams(
            dimension_semantics=("arbitrary",)),
    )
    muid, mts, met, m_uid8, m_exe32, m_ts8, m_et8 = stage_a(uid3, ts3, et3)

    RB = 1000
    GB = n // RB
    assert GB * RB == n
    m_exe_col = m_exe32.reshape(n, 1)
    mexe = pl.pallas_call(
        _exe_kernel,
        grid=(GB,),
        in_specs=[
            pl.BlockSpec((RB, d), lambda b: (b, 0)),
            pl.BlockSpec((RB, 1), lambda b: (b, 0)),
        ],
        out_specs=pl.BlockSpec((RB, d), lambda b: (b, 0)),
        out_shape=jax.ShapeDtypeStruct((n, d), exe_path.dtype),
        compiler_params=pltpu.CompilerParams(
            dimension_semantics=("arbitrary",)),
    )(exe_path, m_exe_col)

    return (muid.reshape(n), mexe, mts.reshape(n), met.reshape(n),
            m_uid8.reshape(n).astype(jnp.bool_),
            m_exe32.reshape(n).astype(jnp.bool_),
            m_ts8.reshape(n).astype(jnp.bool_),
            m_et8.reshape(n).astype(jnp.bool_))


# speedup vs baseline: 1.5666x; 1.5666x over previous
import functools

import jax
import jax.numpy as jnp
from jax.experimental import pallas as pl
from jax.experimental.pallas import tpu as pltpu

_KEYS = (
    (1832780943, 270669613),
    (64467757, 2916123636),
    (2465931498, 255383827),
    (3134548294, 894150801),
)
_THRESH = (2516583, 1677722, 3355444, 838861)

_ROTS = ((13, 15, 26, 6), (17, 29, 16, 24))


def _threefry_bits(cnt_u32, k0, k1):
    ks0 = jnp.uint32(k0)
    ks1 = jnp.uint32(k1)
    ks2 = jnp.uint32((k0 ^ k1 ^ 0x1BD11BDA) & 0xFFFFFFFF)
    ks = (ks0, ks1, ks2)
    x0 = jnp.full_like(cnt_u32, ks0)
    x1 = cnt_u32 + ks1
    for i in range(5):
        rots = _ROTS[i % 2]
        for r in rots:
            x0 = x0 + x1
            x1 = (x1 << jnp.uint32(r)) | (x1 >> jnp.uint32(32 - r))
            x1 = x1 ^ x0
        x0 = x0 + ks[(i + 1) % 3]
        x1 = x1 + ks[(i + 2) % 3] + jnp.uint32(i + 1)
    return x0 ^ x1


def _masks_for(cnt_u32):
    out = []
    for (k0, k1), t in zip(_KEYS, _THRESH):
        bits = _threefry_bits(cnt_u32, k0, k1)
        mant = jnp.right_shift(bits, jnp.uint32(9)).astype(jnp.int32)
        out.append(mant < t)
    return out


def _attrs_kernel(S, L, uid_ref, ts_ref, et_ref,
                  muid_ref, mts_ref, met_ref,
                  m0_ref, m1_ref, m2_ref, m3_ref, mexe_t_ref):
    b = pl.program_id(0)
    shape = (S, L)
    s_io = jax.lax.broadcasted_iota(jnp.int32, shape, 0)
    l_io = jax.lax.broadcasted_iota(jnp.int32, shape, 1)
    j = b * (S * L) + s_io * L + l_io
    m_uid, m_exe, m_ts, m_et = _masks_for(j.astype(jnp.uint32))

    blk = S * L
    uid = uid_ref[...].reshape(shape)
    ts = ts_ref[...].reshape(shape)
    et = et_ref[...].reshape(shape)
    muid_ref[...] = jnp.where(m_uid, 0, uid).reshape(blk)
    mts_ref[...] = jnp.where(m_ts, jnp.float32(0), ts).reshape(blk)
    met_ref[...] = jnp.where(m_et, 0, et).reshape(blk)
    m0_ref[...] = m_uid.reshape(blk)
    m1_ref[...] = m_exe.reshape(blk)
    m2_ref[...] = m_ts.reshape(blk)
    m3_ref[...] = m_et.reshape(blk)
    mexe_t_ref[...] = m_exe.astype(jnp.int32).reshape(mexe_t_ref.shape)


def _exe_kernel(x_ref, m_ref, o_ref):
    m = m_ref[...]
    mcol = m.reshape(m.shape[2], 1)
    o_ref[...] = jnp.where(mcol != 0, jnp.float32(0), x_ref[...])


def kernel(uid, exe_path, timestamp, edge_type):
    n, d = exe_path.shape
    S, L = 8, 1024
    blk = S * L
    G = -(-n // blk)

    RB = 1024
    GB = -(-n // RB)
    tpb = blk // RB

    blk1 = pl.BlockSpec((blk,), lambda b: (b,))
    stage_a = pl.pallas_call(
        functools.partial(_attrs_kernel, S, L),
        grid=(G,),
        in_specs=[blk1, blk1, blk1],
        out_specs=[blk1] * 7 + [pl.BlockSpec((tpb, 1, RB), lambda b: (b, 0, 0))],
        out_shape=[
            jax.ShapeDtypeStruct((n,), uid.dtype),
            jax.ShapeDtypeStruct((n,), timestamp.dtype),
            jax.ShapeDtypeStruct((n,), edge_type.dtype),
            jax.ShapeDtypeStruct((n,), jnp.bool_),
            jax.ShapeDtypeStruct((n,), jnp.bool_),
            jax.ShapeDtypeStruct((n,), jnp.bool_),
            jax.ShapeDtypeStruct((n,), jnp.bool_),
            jax.ShapeDtypeStruct((GB, 1, RB), jnp.int32),
        ],
        compiler_params=pltpu.CompilerParams(
            dimension_semantics=("arbitrary",)),
    )
    (muid, mts, met, m_uid, m_exe, m_ts, m_et,
     m_exe_t) = stage_a(uid, timestamp, edge_type)

    mexe = pl.pallas_call(
        _exe_kernel,
        grid=(GB,),
        in_specs=[
            pl.BlockSpec((RB, d), lambda b: (b, 0)),
            pl.BlockSpec((1, 1, RB), lambda b: (b, 0, 0)),
        ],
        out_specs=pl.BlockSpec((RB, d), lambda b: (b, 0)),
        out_shape=jax.ShapeDtypeStruct((n, d), exe_path.dtype),
        compiler_params=pltpu.CompilerParams(
            dimension_semantics=("arbitrary",)),
    )(exe_path, m_exe_t)

    return muid, mexe, mts, met, m_uid, m_exe, m_ts, m_et
